# baseline (speedup 1.0000x reference)
import functools

import jax
import jax.numpy as jnp
from jax import lax
from jax.experimental import pallas as pl
from jax.experimental.pallas import tpu as pltpu
from jax.experimental.pallas import tpu_sc as plsc

N = 10000
E = 160000
D = 256
DE = 16
H = 4
F = 64
K = 3
NEG = 0.2

NPAD = 10240
CHUNK = 128
NCHUNKS = E // CHUNK
NSUB = 16
ROWS_PER_SUB = NPAD // NSUB


def _proj_body(x_ref, w_ref, o_ref):
    o_ref[...] = jnp.dot(x_ref[...], w_ref[...],
                         preferred_element_type=jnp.float32)


def _dense_proj(x, w_cat, block_rows):
    R, Dk = x.shape
    C = w_cat.shape[1]
    grid = (R // block_rows,)
    return pl.pallas_call(
        _proj_body,
        grid=grid,
        in_specs=[
            pl.BlockSpec((block_rows, Dk), lambda i: (i, 0)),
            pl.BlockSpec((Dk, C), lambda i: (0, 0)),
        ],
        out_specs=pl.BlockSpec((block_rows, C), lambda i: (i, 0)),
        out_shape=jax.ShapeDtypeStruct((R, C), jnp.float32),
    )(x, w_cat)


def _combine_body(proj_ref, b_ref, h1l, h1h, h2l, h2h, h3l, h3h,
                  scl_ref, off_ref, pos_ref, al_ref, ar_ref, out_ref):
    f32 = jnp.float32
    hp = jax.lax.Precision.HIGHEST
    rows = lax.broadcasted_iota(jnp.int32, (256, 4), 0) // 64
    cols = lax.broadcasted_iota(jnp.int32, (256, 4), 1)
    m4 = jnp.where(rows == cols, 1.0, 0.0).astype(f32)
    m4avg = m4 * (1.0 / 64.0)
    e4 = m4.T

    def ft(h, k):
        mean4 = jnp.dot(h, m4avg, precision=hp, preferred_element_type=f32)
        ctr = h - jnp.dot(mean4, e4, precision=hp, preferred_element_type=f32)
        var4 = jnp.dot(ctr * ctr, m4avg, precision=hp, preferred_element_type=f32) + 1e-9
        rsE = jnp.dot(lax.rsqrt(var4), e4, precision=hp, preferred_element_type=f32)
        return ctr * rsE * scl_ref[k][None, :] + off_ref[k][None, :] + pos_ref[k][None, :]

    h0 = ft(proj_ref[:, 0:256], 0)
    lrow = al_ref[0][None, :]
    rrow = ar_ref[0][None, :]
    al4 = jnp.dot(h0 * lrow, m4, precision=hp, preferred_element_type=f32)
    fts = []
    w4s = []
    for k, (lo, hi) in enumerate(((h1l, h1h), (h2l, h2h), (h3l, h3h))):
        h = jnp.concatenate([lo[...], hi[...]], axis=1)
        f = ft(h, k + 1)
        fts.append(f)
        s = jnp.dot(f * rrow, m4, precision=hp, preferred_element_type=f32) + al4
        s = jnp.where(s >= 0, s, NEG * s)
        w4s.append(jnp.exp(s))
    den = w4s[0] + w4s[1] + w4s[2]
    acc = proj_ref[:, 256:512] + b_ref[...]
    for k in range(3):
        wE = jnp.dot(w4s[k] / den, e4, precision=hp, preferred_element_type=f32)
        acc = acc + fts[k] * wE
    out_ref[...] = acc


def _combine_tc(proj, b_dst, h2s, scl, off, pos, alf, arf):
    B = 1024
    grid = (NPAD // B,)
    hspec_lo = pl.BlockSpec((B, 128), lambda i: (i, 0))
    hspec_hi = pl.BlockSpec((B, 128), lambda i: (i + NPAD // B, 0))
    full = lambda shape: pl.BlockSpec(shape, lambda i: tuple(0 for _ in shape))
    return pl.pallas_call(
        _combine_body,
        grid=grid,
        in_specs=[
            pl.BlockSpec((B, 640), lambda i: (i, 0)),
            full((1, 256)),
            hspec_lo, hspec_hi, hspec_lo, hspec_hi, hspec_lo, hspec_hi,
            full((4, 256)), full((4, 256)), full((4, 256)),
            full((1, 256)), full((1, 256)),
        ],
        out_specs=pl.BlockSpec((B, 256), lambda i: (i, 0)),
        out_shape=jax.ShapeDtypeStruct((NPAD, 256), jnp.float32),
    )(proj, b_dst, h2s[0], h2s[0], h2s[1], h2s[1], h2s[2], h2s[2],
      scl, off, pos, alf, arf)


def _edge_body(asrc_t, adst_t, ae_t, src_g, dst_g, ex_o,
               sidx_v, didx_v, as_v, ad_v, ae_v, ex16_v, sem):
    c = lax.axis_index("c")
    s = lax.axis_index("s")
    w = c * NSUB + s
    nchunks = (NCHUNKS - w + 31) // 32

    def _chunk(i, _):
        g = w + i * 32
        base = g * CHUNK
        pltpu.sync_copy(src_g.at[pl.ds(base, CHUNK)], sidx_v)
        pltpu.sync_copy(dst_g.at[pl.ds(base, CHUNK)], didx_v)
        pltpu.sync_copy(ae_t.at[pl.ds(base, CHUNK)], ae_v)
        cp1 = pltpu.async_copy(asrc_t.at[sidx_v], as_v, sem)
        cp2 = pltpu.async_copy(adst_t.at[didx_v], ad_v, sem)
        cp1.wait()
        cp2.wait()

        def _row(r, _):
            x = as_v[r, pl.ds(0, 16)] + ad_v[r, pl.ds(0, 16)] + ae_v[r, pl.ds(0, 16)]
            x = jnp.where(x >= 0, x, NEG * x)
            ex16_v[r, pl.ds(0, 16)] = jnp.exp(x)
            return _
        lax.fori_loop(0, CHUNK, _row, None, unroll=4)
        pltpu.sync_copy(ex16_v, ex_o.at[pl.ds(base, CHUNK)])
        return _

    lax.fori_loop(0, nchunks, _chunk, None)


@functools.cache
def _edge_sc_kernel():
    return functools.partial(
        pl.kernel,
        mesh=plsc.VectorSubcoreMesh(core_axis_name="c", subcore_axis_name="s"),
        out_type=jax.ShapeDtypeStruct((E, 16), jnp.float32),
        scratch_types=[
            pltpu.VMEM((CHUNK,), jnp.int32),
            pltpu.VMEM((CHUNK,), jnp.int32),
            pltpu.VMEM((CHUNK, 128), jnp.float32),
            pltpu.VMEM((CHUNK, 128), jnp.float32),
            pltpu.VMEM((CHUNK, 16), jnp.float32),
            pltpu.VMEM((CHUNK, 16), jnp.float32),
            pltpu.SemaphoreType.DMA,
        ],
    )(_edge_body)


def _acoef_body(ex_t, sd_t, ss_t, src_g, dst_g, a_o,
                sidx_v, didx_v, ex_v, sd_v, ss_v, a_v, sem):
    c = lax.axis_index("c")
    s = lax.axis_index("s")
    w = c * NSUB + s
    nchunks = (NCHUNKS - w + 31) // 32

    def _chunk(i, _):
        g = w + i * 32
        base = g * CHUNK
        pltpu.sync_copy(src_g.at[pl.ds(base, CHUNK)], sidx_v)
        pltpu.sync_copy(dst_g.at[pl.ds(base, CHUNK)], didx_v)
        pltpu.sync_copy(ex_t.at[pl.ds(base, CHUNK)], ex_v)
        cp1 = pltpu.async_copy(sd_t.at[didx_v], sd_v, sem)
        cp2 = pltpu.async_copy(ss_t.at[sidx_v], ss_v, sem)
        cp1.wait()
        cp2.wait()

        def _row(r, _):
            exv = ex_v[r, pl.ds(0, 16)]
            ad = jnp.maximum(exv / (sd_v[r, pl.ds(0, 16)] + 1e-16), 1e-9)
            asv = jnp.maximum(exv / (ss_v[r, pl.ds(0, 16)] + 1e-16), 1e-9)
            p = ad * asv
            iv = lax.bitcast_convert_type(p, jnp.int32)
            y = lax.bitcast_convert_type(
                jnp.full((16,), 0x5F3759DF, jnp.int32) - (iv >> 1), jnp.float32)
            for _i in range(3):
                y = y * (1.5 - 0.5 * p * y * y)
            a_v[r, pl.ds(0, 16)] = p * y
            return _
        lax.fori_loop(0, CHUNK, _row, None, unroll=4)
        pltpu.sync_copy(a_v, a_o.at[pl.ds(base, CHUNK)])
        return _

    lax.fori_loop(0, nchunks, _chunk, None)


@functools.cache
def _acoef_sc_kernel():
    return functools.partial(
        pl.kernel,
        mesh=plsc.VectorSubcoreMesh(core_axis_name="c", subcore_axis_name="s"),
        out_type=jax.ShapeDtypeStruct((E, 16), jnp.float32),
        scratch_types=[
            pltpu.VMEM((CHUNK,), jnp.int32),
            pltpu.VMEM((CHUNK,), jnp.int32),
            pltpu.VMEM((CHUNK, 16), jnp.float32),
            pltpu.VMEM((CHUNK, 128), jnp.float32),
            pltpu.VMEM((CHUNK, 128), jnp.float32),
            pltpu.VMEM((CHUNK, 16), jnp.float32),
            pltpu.SemaphoreType.DMA,
        ],
    )(_acoef_body)


def _hop_body(h2, a_t, src_g, dst_g, out2, sidx_v, didx_v, a_v, rows_v, zbuf,
              acc, sem):
    c = lax.axis_index("c")
    s = lax.axis_index("s")

    def _zrow(i, _):
        for j in range(8):
            zbuf[i, pl.ds(j * 16, 16)] = jnp.zeros((16,), jnp.float32)
        return _
    lax.fori_loop(0, 80, _zrow, None)
    for r in range(ROWS_PER_SUB // 80):
        pltpu.sync_copy(zbuf, acc.at[pl.ds(s * ROWS_PER_SUB + r * 80, 80)])
    plsc.subcore_barrier()

    nchunks = (NCHUNKS - s + NSUB - 1) // NSUB

    def _chunk(i, _):
        g = s + i * NSUB
        base = g * CHUNK
        pltpu.sync_copy(src_g.at[pl.ds(base, CHUNK)], sidx_v)
        pltpu.sync_copy(dst_g.at[pl.ds(base, CHUNK)], didx_v)
        pltpu.sync_copy(a_t.at[pl.ds(base, CHUNK)], a_v)
        for j in range(CHUNK // 16):
            sidx_v[pl.ds(j * 16, 16)] = sidx_v[pl.ds(j * 16, 16)] + c * NPAD
        pltpu.async_copy(h2.at[sidx_v], rows_v, sem).wait()

        def _scale(e, _):
            blk = a_v[e, pl.ds(0, 16)]
            s0 = blk.at[jnp.full((16,), 2 * c, jnp.int32)].get(
                mode="promise_in_bounds")
            s1 = blk.at[jnp.full((16,), 2 * c + 1, jnp.int32)].get(
                mode="promise_in_bounds")
            for j in range(4):
                rows_v[e, pl.ds(j * 16, 16)] = rows_v[e, pl.ds(j * 16, 16)] * s0
            for j in range(4, 8):
                rows_v[e, pl.ds(j * 16, 16)] = rows_v[e, pl.ds(j * 16, 16)] * s1
            return _
        lax.fori_loop(0, CHUNK, _scale, None, unroll=4)
        pltpu.sync_copy(rows_v, acc.at[didx_v], add=True)
        return _

    lax.fori_loop(0, nchunks, _chunk, None)
    plsc.subcore_barrier()
    pltpu.sync_copy(acc.at[pl.ds(s * ROWS_PER_SUB, ROWS_PER_SUB)],
                    out2.at[pl.ds(c * NPAD + s * ROWS_PER_SUB, ROWS_PER_SUB)])


@functools.cache
def _hop_sc_kernel():
    return functools.partial(
        pl.kernel,
        mesh=plsc.VectorSubcoreMesh(core_axis_name="c", subcore_axis_name="s"),
        out_type=jax.ShapeDtypeStruct((2 * NPAD, 128), jnp.float32),
        scratch_types=[
            pltpu.VMEM((CHUNK,), jnp.int32),
            pltpu.VMEM((CHUNK,), jnp.int32),
            pltpu.VMEM((CHUNK, 16), jnp.float32),
            pltpu.VMEM((CHUNK, 128), jnp.float32),
            pltpu.VMEM((80, 128), jnp.float32),
            pltpu.VMEM_SHARED((NPAD, 128), jnp.float32),
            pltpu.SemaphoreType.DMA,
        ],
    )(_hop_body)


def leaky_relu(x):
    return jnp.where(x >= 0, x, NEG * x)


def kernel(feat_src, edge_index, feat_edge, W_src, W_dst, b_dst, W_attn_src,
           W_attn_dst, W_attn_edge, scale, offset, position_emb, hop_attn_l,
           hop_attn_r):
    src = edge_index[0]
    dst = edge_index[1]

    w_cat = jnp.concatenate([W_src, W_dst, W_attn_src, W_attn_dst], axis=1)
    w_cat = jnp.pad(w_cat, ((0, 0), (0, 640 - w_cat.shape[1])))
    feat_pad = jnp.pad(feat_src, ((0, NPAD - N), (0, 0)))
    proj = _dense_proj(feat_pad, w_cat, block_rows=1024)

    w_e = jnp.pad(W_attn_edge, ((0, 0), (0, 16 - H)))
    ae16 = _dense_proj(feat_edge, w_e, block_rows=8000)

    asrc128 = jnp.pad(proj[:, 512:512 + H], ((0, NPAD - N), (0, 124)))
    adst128 = jnp.pad(proj[:, 512 + H:512 + 2 * H], ((0, NPAD - N), (0, 124)))
    ex16 = _edge_sc_kernel()(asrc128, adst128, ae16, src, dst)
    ones2 = jnp.ones((2 * NPAD, 128), jnp.float32)
    zidx = jnp.zeros((E,), jnp.int32)
    s2d = _hop_sc_kernel()(ones2, ex16, zidx, dst)
    s2s = _hop_sc_kernel()(ones2, ex16, zidx, src)
    sd128 = jnp.pad(jnp.stack([s2d[:NPAD, 0], s2d[:NPAD, 64],
                               s2d[NPAD:, 0], s2d[NPAD:, 64]], axis=1),
                    ((0, 0), (0, 124)))
    ss128 = jnp.pad(jnp.stack([s2s[:NPAD, 0], s2s[:NPAD, 64],
                               s2s[NPAD:, 0], s2s[NPAD:, 64]], axis=1),
                    ((0, 0), (0, 124)))
    a16 = _acoef_sc_kernel()(ex16, sd128, ss128, src, dst)

    fc_pad = proj[:, :256]
    h2 = jnp.concatenate([fc_pad[:, :128], fc_pad[:, 128:]], axis=0)
    h2s = []
    for k in range(K):
        h2 = _hop_sc_kernel()(h2, a16, src, dst)
        h2s.append(h2)

    rst_pad = _combine_tc(
        proj, b_dst.reshape(1, 256), h2s,
        scale.reshape(K + 1, H * F), offset.reshape(K + 1, H * F),
        position_emb.reshape(K + 1, H * F),
        hop_attn_l.reshape(1, H * F), hop_attn_r.reshape(1, H * F))
    return rst_pad[:N].reshape(N, H, F)


# speedup vs baseline: 4.3894x; 4.3894x over previous
import functools

import jax
import jax.numpy as jnp
from jax import lax
from jax.experimental import pallas as pl
from jax.experimental.pallas import tpu as pltpu
from jax.experimental.pallas import tpu_sc as plsc

N = 10000
E = 160000
D = 256
DE = 16
H = 4
F = 64
K = 3
NEG = 0.2

NPAD = 10240
CHUNK = 128
NCHUNKS = E // CHUNK
NSUB = 16
ROWS_PER_SUB = NPAD // NSUB


def _proj_body(x_ref, w_ref, o_ref):
    o_ref[...] = jnp.dot(x_ref[...], w_ref[...],
                         preferred_element_type=jnp.float32)


def _dense_proj(x, w_cat, block_rows):
    R, Dk = x.shape
    C = w_cat.shape[1]
    grid = (R // block_rows,)
    return pl.pallas_call(
        _proj_body,
        grid=grid,
        in_specs=[
            pl.BlockSpec((block_rows, Dk), lambda i: (i, 0)),
            pl.BlockSpec((Dk, C), lambda i: (0, 0)),
        ],
        out_specs=pl.BlockSpec((block_rows, C), lambda i: (i, 0)),
        out_shape=jax.ShapeDtypeStruct((R, C), jnp.float32),
    )(x, w_cat)


def _combine_body(proj_ref, b_ref, h1l, h1h, h2l, h2h, h3l, h3h,
                  scl_ref, off_ref, pos_ref, al_ref, ar_ref, out_ref):
    f32 = jnp.float32
    hp = jax.lax.Precision.HIGHEST
    rows = lax.broadcasted_iota(jnp.int32, (256, 4), 0) // 64
    cols = lax.broadcasted_iota(jnp.int32, (256, 4), 1)
    m4 = jnp.where(rows == cols, 1.0, 0.0).astype(f32)
    m4avg = m4 * (1.0 / 64.0)
    e4 = m4.T

    def ft(h, k):
        mean4 = jnp.dot(h, m4avg, precision=hp, preferred_element_type=f32)
        ctr = h - jnp.dot(mean4, e4, precision=hp, preferred_element_type=f32)
        var4 = jnp.dot(ctr * ctr, m4avg, precision=hp, preferred_element_type=f32) + 1e-9
        rsE = jnp.dot(lax.rsqrt(var4), e4, precision=hp, preferred_element_type=f32)
        return ctr * rsE * scl_ref[k][None, :] + off_ref[k][None, :] + pos_ref[k][None, :]

    h0 = ft(proj_ref[:, 0:256], 0)
    lrow = al_ref[0][None, :]
    rrow = ar_ref[0][None, :]
    al4 = jnp.dot(h0 * lrow, m4, precision=hp, preferred_element_type=f32)
    fts = []
    w4s = []
    for k, (lo, hi) in enumerate(((h1l, h1h), (h2l, h2h), (h3l, h3h))):
        h = jnp.concatenate([lo[...], hi[...]], axis=1)
        f = ft(h, k + 1)
        fts.append(f)
        s = jnp.dot(f * rrow, m4, precision=hp, preferred_element_type=f32) + al4
        s = jnp.where(s >= 0, s, NEG * s)
        w4s.append(jnp.exp(s))
    den = w4s[0] + w4s[1] + w4s[2]
    acc = proj_ref[:, 256:512] + b_ref[...]
    for k in range(3):
        wE = jnp.dot(w4s[k] / den, e4, precision=hp, preferred_element_type=f32)
        acc = acc + fts[k] * wE
    out_ref[...] = acc


def _combine_tc(proj, b_dst, h2s, scl, off, pos, alf, arf):
    B = 1024
    grid = (NPAD // B,)
    hspec_lo = pl.BlockSpec((B, 128), lambda i: (i, 0))
    hspec_hi = pl.BlockSpec((B, 128), lambda i: (i + NPAD // B, 0))
    full = lambda shape: pl.BlockSpec(shape, lambda i: tuple(0 for _ in shape))
    return pl.pallas_call(
        _combine_body,
        grid=grid,
        in_specs=[
            pl.BlockSpec((B, 640), lambda i: (i, 0)),
            full((1, 256)),
            hspec_lo, hspec_hi, hspec_lo, hspec_hi, hspec_lo, hspec_hi,
            full((4, 256)), full((4, 256)), full((4, 256)),
            full((1, 256)), full((1, 256)),
        ],
        out_specs=pl.BlockSpec((B, 256), lambda i: (i, 0)),
        out_shape=jax.ShapeDtypeStruct((NPAD, 256), jnp.float32),
    )(proj, b_dst, h2s[0], h2s[0], h2s[1], h2s[1], h2s[2], h2s[2],
      scl, off, pos, alf, arf)


def _edge_body(asrc_t, adst_t, ae_t, src_g, dst_g, ex_o,
               sidx_v, didx_v, as_v, ad_v, ae_v, ex16_v, sem):
    c = lax.axis_index("c")
    s = lax.axis_index("s")
    w = c * NSUB + s
    nchunks = (NCHUNKS - w + 31) // 32

    def _chunk(i, _):
        g = w + i * 32
        base = g * CHUNK
        pltpu.sync_copy(src_g.at[pl.ds(base, CHUNK)], sidx_v)
        pltpu.sync_copy(dst_g.at[pl.ds(base, CHUNK)], didx_v)
        pltpu.sync_copy(ae_t.at[pl.ds(base, CHUNK)], ae_v)
        cp1 = pltpu.async_copy(asrc_t.at[sidx_v], as_v, sem)
        cp2 = pltpu.async_copy(adst_t.at[didx_v], ad_v, sem)
        cp1.wait()
        cp2.wait()

        def _row(r, _):
            x = as_v[r, pl.ds(0, 16)] + ad_v[r, pl.ds(0, 16)] + ae_v[r, pl.ds(0, 16)]
            x = jnp.where(x >= 0, x, NEG * x)
            ex16_v[r, pl.ds(0, 16)] = jnp.exp(x)
            return _
        lax.fori_loop(0, CHUNK, _row, None, unroll=4)
        pltpu.sync_copy(ex16_v, ex_o.at[pl.ds(base, CHUNK)])
        return _

    lax.fori_loop(0, nchunks, _chunk, None)


@functools.cache
def _edge_sc_kernel():
    return functools.partial(
        pl.kernel,
        mesh=plsc.VectorSubcoreMesh(core_axis_name="c", subcore_axis_name="s"),
        out_type=jax.ShapeDtypeStruct((E, 16), jnp.float32),
        scratch_types=[
            pltpu.VMEM((CHUNK,), jnp.int32),
            pltpu.VMEM((CHUNK,), jnp.int32),
            pltpu.VMEM((CHUNK, 128), jnp.float32),
            pltpu.VMEM((CHUNK, 128), jnp.float32),
            pltpu.VMEM((CHUNK, 16), jnp.float32),
            pltpu.VMEM((CHUNK, 16), jnp.float32),
            pltpu.SemaphoreType.DMA,
        ],
    )(_edge_body)


def _acoef_body(ex_t, sd_t, ss_t, src_g, dst_g, a_o,
                sidx_v, didx_v, ex_v, sd_v, ss_v, a_v, sem):
    c = lax.axis_index("c")
    s = lax.axis_index("s")
    w = c * NSUB + s
    nchunks = (NCHUNKS - w + 31) // 32

    def _chunk(i, _):
        g = w + i * 32
        base = g * CHUNK
        pltpu.sync_copy(src_g.at[pl.ds(base, CHUNK)], sidx_v)
        pltpu.sync_copy(dst_g.at[pl.ds(base, CHUNK)], didx_v)
        pltpu.sync_copy(ex_t.at[pl.ds(base, CHUNK)], ex_v)
        cp1 = pltpu.async_copy(sd_t.at[didx_v], sd_v, sem)
        cp2 = pltpu.async_copy(ss_t.at[sidx_v], ss_v, sem)
        cp1.wait()
        cp2.wait()

        def _row(r, _):
            exv = ex_v[r, pl.ds(0, 16)]
            ad = jnp.maximum(exv / (sd_v[r, pl.ds(0, 16)] + 1e-16), 1e-9)
            asv = jnp.maximum(exv / (ss_v[r, pl.ds(0, 16)] + 1e-16), 1e-9)
            p = ad * asv
            iv = lax.bitcast_convert_type(p, jnp.int32)
            y = lax.bitcast_convert_type(
                jnp.full((16,), 0x5F3759DF, jnp.int32) - (iv >> 1), jnp.float32)
            for _i in range(3):
                y = y * (1.5 - 0.5 * p * y * y)
            a_v[r, pl.ds(0, 16)] = p * y
            return _
        lax.fori_loop(0, CHUNK, _row, None, unroll=4)
        pltpu.sync_copy(a_v, a_o.at[pl.ds(base, CHUNK)])
        return _

    lax.fori_loop(0, nchunks, _chunk, None)


@functools.cache
def _acoef_sc_kernel():
    return functools.partial(
        pl.kernel,
        mesh=plsc.VectorSubcoreMesh(core_axis_name="c", subcore_axis_name="s"),
        out_type=jax.ShapeDtypeStruct((E, 16), jnp.float32),
        scratch_types=[
            pltpu.VMEM((CHUNK,), jnp.int32),
            pltpu.VMEM((CHUNK,), jnp.int32),
            pltpu.VMEM((CHUNK, 16), jnp.float32),
            pltpu.VMEM((CHUNK, 128), jnp.float32),
            pltpu.VMEM((CHUNK, 128), jnp.float32),
            pltpu.VMEM((CHUNK, 16), jnp.float32),
            pltpu.SemaphoreType.DMA,
        ],
    )(_acoef_body)


def _hop_body(h2, a_t, src_g, dst_g, out2, sidx_v, didx_v, a_v, rows_v, zbuf,
              acc, sem):
    c = lax.axis_index("c")
    s = lax.axis_index("s")

    def _zrow(i, _):
        for j in range(8):
            zbuf[i, pl.ds(j * 16, 16)] = jnp.zeros((16,), jnp.float32)
        return _
    lax.fori_loop(0, 80, _zrow, None)
    for r in range(ROWS_PER_SUB // 80):
        pltpu.sync_copy(zbuf, acc.at[pl.ds(s * ROWS_PER_SUB + r * 80, 80)])
    plsc.subcore_barrier()

    nchunks = (NCHUNKS - s + NSUB - 1) // NSUB

    def _chunk(i, _):
        g = s + i * NSUB
        base = g * CHUNK
        pltpu.sync_copy(src_g.at[pl.ds(base, CHUNK)], sidx_v)
        pltpu.sync_copy(dst_g.at[pl.ds(base, CHUNK)], didx_v)
        pltpu.sync_copy(a_t.at[pl.ds(base, CHUNK)], a_v)
        for j in range(CHUNK // 16):
            sidx_v[pl.ds(j * 16, 16)] = sidx_v[pl.ds(j * 16, 16)] + c * NPAD
        pltpu.async_copy(h2.at[sidx_v], rows_v, sem).wait()

        def _scale(e, _):
            blk = a_v[e, pl.ds(0, 16)]
            s0 = blk.at[jnp.full((16,), 2 * c, jnp.int32)].get(
                mode="promise_in_bounds")
            s1 = blk.at[jnp.full((16,), 2 * c + 1, jnp.int32)].get(
                mode="promise_in_bounds")
            for j in range(4):
                rows_v[e, pl.ds(j * 16, 16)] = rows_v[e, pl.ds(j * 16, 16)] * s0
            for j in range(4, 8):
                rows_v[e, pl.ds(j * 16, 16)] = rows_v[e, pl.ds(j * 16, 16)] * s1
            return _
        lax.fori_loop(0, CHUNK, _scale, None, unroll=4)
        pltpu.sync_copy(rows_v, acc.at[didx_v], add=True)
        return _

    lax.fori_loop(0, nchunks, _chunk, None)
    plsc.subcore_barrier()
    pltpu.sync_copy(acc.at[pl.ds(s * ROWS_PER_SUB, ROWS_PER_SUB)],
                    out2.at[pl.ds(c * NPAD + s * ROWS_PER_SUB, ROWS_PER_SUB)])


@functools.cache
def _hop_sc_kernel():
    return functools.partial(
        pl.kernel,
        mesh=plsc.VectorSubcoreMesh(core_axis_name="c", subcore_axis_name="s"),
        out_type=jax.ShapeDtypeStruct((2 * NPAD, 128), jnp.float32),
        scratch_types=[
            pltpu.VMEM((CHUNK,), jnp.int32),
            pltpu.VMEM((CHUNK,), jnp.int32),
            pltpu.VMEM((CHUNK, 16), jnp.float32),
            pltpu.VMEM((CHUNK, 128), jnp.float32),
            pltpu.VMEM((80, 128), jnp.float32),
            pltpu.VMEM_SHARED((NPAD, 128), jnp.float32),
            pltpu.SemaphoreType.DMA,
        ],
    )(_hop_body)


def leaky_relu(x):
    return jnp.where(x >= 0, x, NEG * x)


def kernel(feat_src, edge_index, feat_edge, W_src, W_dst, b_dst, W_attn_src,
           W_attn_dst, W_attn_edge, scale, offset, position_emb, hop_attn_l,
           hop_attn_r):
    src = edge_index[0]
    dst = edge_index[1]

    w_cat = jnp.concatenate([W_src, W_dst, W_attn_src, W_attn_dst], axis=1)
    w_cat = jnp.pad(w_cat, ((0, 0), (0, 640 - w_cat.shape[1])))
    feat_pad = jnp.pad(feat_src, ((0, NPAD - N), (0, 0)))
    proj = _dense_proj(feat_pad, w_cat, block_rows=1024)

    w_e = jnp.pad(W_attn_edge, ((0, 0), (0, 16 - H)))
    ae16 = _dense_proj(feat_edge, w_e, block_rows=8000)

    asrc128 = jnp.pad(proj[:, 512:512 + H], ((0, NPAD - N), (0, 124)))
    adst128 = jnp.pad(proj[:, 512 + H:512 + 2 * H], ((0, NPAD - N), (0, 124)))
    ex16 = _edge_sc_kernel()(asrc128, adst128, ae16, src, dst)
    ones2 = jnp.ones((2 * NPAD, 128), jnp.float32)
    s2d = _hop_sc_kernel()(ones2, ex16, src, dst)
    s2s = _hop_sc_kernel()(ones2, ex16, dst, src)
    sd128 = jnp.pad(jnp.stack([s2d[:NPAD, 0], s2d[:NPAD, 64],
                               s2d[NPAD:, 0], s2d[NPAD:, 64]], axis=1),
                    ((0, 0), (0, 124)))
    ss128 = jnp.pad(jnp.stack([s2s[:NPAD, 0], s2s[:NPAD, 64],
                               s2s[NPAD:, 0], s2s[NPAD:, 64]], axis=1),
                    ((0, 0), (0, 124)))
    a16 = _acoef_sc_kernel()(ex16, sd128, ss128, src, dst)

    fc_pad = proj[:, :256]
    h2 = jnp.concatenate([fc_pad[:, :128], fc_pad[:, 128:]], axis=0)
    h2s = []
    for k in range(K):
        h2 = _hop_sc_kernel()(h2, a16, src, dst)
        h2s.append(h2)

    rst_pad = _combine_tc(
        proj, b_dst.reshape(1, 256), h2s,
        scale.reshape(K + 1, H * F), offset.reshape(K + 1, H * F),
        position_emb.reshape(K + 1, H * F),
        hop_attn_l.reshape(1, H * F), hop_attn_r.reshape(1, H * F))
    return rst_pad[:N].reshape(N, H, F)


# speedup vs baseline: 4.9885x; 1.1365x over previous
import functools

import jax
import jax.numpy as jnp
from jax import lax
from jax.experimental import pallas as pl
from jax.experimental.pallas import tpu as pltpu
from jax.experimental.pallas import tpu_sc as plsc

N = 10000
E = 160000
D = 256
DE = 16
H = 4
F = 64
K = 3
NEG = 0.2

NPAD = 10240
CHUNK = 128
NCHUNKS = E // CHUNK
NSUB = 16
ROWS_PER_SUB = NPAD // NSUB


def _proj_body(x_ref, w_ref, o_ref):
    o_ref[...] = jnp.dot(x_ref[...], w_ref[...],
                         preferred_element_type=jnp.float32)


def _dense_proj(x, w_cat, block_rows):
    R, Dk = x.shape
    C = w_cat.shape[1]
    grid = (R // block_rows,)
    return pl.pallas_call(
        _proj_body,
        grid=grid,
        in_specs=[
            pl.BlockSpec((block_rows, Dk), lambda i: (i, 0)),
            pl.BlockSpec((Dk, C), lambda i: (0, 0)),
        ],
        out_specs=pl.BlockSpec((block_rows, C), lambda i: (i, 0)),
        out_shape=jax.ShapeDtypeStruct((R, C), jnp.float32),
    )(x, w_cat)


def _combine_body(proj_ref, b_ref, h1l, h1h, h2l, h2h, h3l, h3h,
                  scl_ref, off_ref, pos_ref, al_ref, ar_ref, out_ref):
    f32 = jnp.float32
    hp = jax.lax.Precision.HIGHEST
    rows = lax.broadcasted_iota(jnp.int32, (256, 4), 0) // 64
    cols = lax.broadcasted_iota(jnp.int32, (256, 4), 1)
    m4 = jnp.where(rows == cols, 1.0, 0.0).astype(f32)
    m4avg = m4 * (1.0 / 64.0)
    e4 = m4.T

    def ft(h, k):
        mean4 = jnp.dot(h, m4avg, precision=hp, preferred_element_type=f32)
        ctr = h - jnp.dot(mean4, e4, precision=hp, preferred_element_type=f32)
        var4 = jnp.dot(ctr * ctr, m4avg, precision=hp, preferred_element_type=f32) + 1e-9
        rsE = jnp.dot(lax.rsqrt(var4), e4, precision=hp, preferred_element_type=f32)
        return ctr * rsE * scl_ref[k][None, :] + off_ref[k][None, :] + pos_ref[k][None, :]

    h0 = ft(proj_ref[:, 0:256], 0)
    lrow = al_ref[0][None, :]
    rrow = ar_ref[0][None, :]
    al4 = jnp.dot(h0 * lrow, m4, precision=hp, preferred_element_type=f32)
    fts = []
    w4s = []
    for k, (lo, hi) in enumerate(((h1l, h1h), (h2l, h2h), (h3l, h3h))):
        h = jnp.concatenate([lo[...], hi[...]], axis=1)
        f = ft(h, k + 1)
        fts.append(f)
        s = jnp.dot(f * rrow, m4, precision=hp, preferred_element_type=f32) + al4
        s = jnp.where(s >= 0, s, NEG * s)
        w4s.append(jnp.exp(s))
    den = w4s[0] + w4s[1] + w4s[2]
    acc = proj_ref[:, 256:512] + b_ref[...]
    for k in range(3):
        wE = jnp.dot(w4s[k] / den, e4, precision=hp, preferred_element_type=f32)
        acc = acc + fts[k] * wE
    out_ref[...] = acc


def _combine_tc(proj, b_dst, h2s, scl, off, pos, alf, arf):
    B = 1024
    grid = (NPAD // B,)
    hspec_lo = pl.BlockSpec((B, 128), lambda i: (i, 0))
    hspec_hi = pl.BlockSpec((B, 128), lambda i: (i + NPAD // B, 0))
    full = lambda shape: pl.BlockSpec(shape, lambda i: tuple(0 for _ in shape))
    return pl.pallas_call(
        _combine_body,
        grid=grid,
        in_specs=[
            pl.BlockSpec((B, 640), lambda i: (i, 0)),
            full((1, 256)),
            hspec_lo, hspec_hi, hspec_lo, hspec_hi, hspec_lo, hspec_hi,
            full((4, 256)), full((4, 256)), full((4, 256)),
            full((1, 256)), full((1, 256)),
        ],
        out_specs=pl.BlockSpec((B, 256), lambda i: (i, 0)),
        out_shape=jax.ShapeDtypeStruct((NPAD, 256), jnp.float32),
    )(proj, b_dst, h2s[0], h2s[0], h2s[1], h2s[1], h2s[2], h2s[2],
      scl, off, pos, alf, arf)


def _edge_body(asrc_t, adst_t, ae_t, src_g, dst_g, ex_o,
               sidx_v, didx_v, as_v, ad_v, ae_v, ex16_v, sem):
    c = lax.axis_index("c")
    s = lax.axis_index("s")
    w = c * NSUB + s
    nchunks = (NCHUNKS - w + 31) // 32

    def _chunk(i, _):
        g = w + i * 32
        base = g * CHUNK
        pltpu.sync_copy(src_g.at[pl.ds(base, CHUNK)], sidx_v)
        pltpu.sync_copy(dst_g.at[pl.ds(base, CHUNK)], didx_v)
        pltpu.sync_copy(ae_t.at[pl.ds(base, CHUNK)], ae_v)
        cp1 = pltpu.async_copy(asrc_t.at[sidx_v], as_v, sem)
        cp2 = pltpu.async_copy(adst_t.at[didx_v], ad_v, sem)
        cp1.wait()
        cp2.wait()

        def _row(r, _):
            x = as_v[r, pl.ds(0, 16)] + ad_v[r, pl.ds(0, 16)] + ae_v[r, pl.ds(0, 16)]
            x = jnp.where(x >= 0, x, NEG * x)
            ex16_v[r, pl.ds(0, 16)] = jnp.exp(x)
            return _
        lax.fori_loop(0, CHUNK, _row, None)
        pltpu.sync_copy(ex16_v, ex_o.at[pl.ds(base, CHUNK)])
        return _

    lax.fori_loop(0, nchunks, _chunk, None)


@functools.cache
def _edge_sc_kernel():
    return functools.partial(
        pl.kernel,
        mesh=plsc.VectorSubcoreMesh(core_axis_name="c", subcore_axis_name="s"),
        out_type=jax.ShapeDtypeStruct((E, 16), jnp.float32),
        scratch_types=[
            pltpu.VMEM((CHUNK,), jnp.int32),
            pltpu.VMEM((CHUNK,), jnp.int32),
            pltpu.VMEM((CHUNK, 128), jnp.float32),
            pltpu.VMEM((CHUNK, 128), jnp.float32),
            pltpu.VMEM((CHUNK, 16), jnp.float32),
            pltpu.VMEM((CHUNK, 16), jnp.float32),
            pltpu.SemaphoreType.DMA,
        ],
    )(_edge_body)


def _acoef_body(ex_t, sd_t, ss_t, src_g, dst_g, a_o,
                sidx_v, didx_v, ex_v, sd_v, ss_v, a_v, sem):
    c = lax.axis_index("c")
    s = lax.axis_index("s")
    w = c * NSUB + s
    nchunks = (NCHUNKS - w + 31) // 32

    def _chunk(i, _):
        g = w + i * 32
        base = g * CHUNK
        pltpu.sync_copy(src_g.at[pl.ds(base, CHUNK)], sidx_v)
        pltpu.sync_copy(dst_g.at[pl.ds(base, CHUNK)], didx_v)
        pltpu.sync_copy(ex_t.at[pl.ds(base, CHUNK)], ex_v)
        cp1 = pltpu.async_copy(sd_t.at[didx_v], sd_v, sem)
        cp2 = pltpu.async_copy(ss_t.at[sidx_v], ss_v, sem)
        cp1.wait()
        cp2.wait()

        def _row(r, _):
            exv = ex_v[r, pl.ds(0, 16)]
            ad = jnp.maximum(exv / (sd_v[r, pl.ds(0, 16)] + 1e-16), 1e-9)
            asv = jnp.maximum(exv / (ss_v[r, pl.ds(0, 16)] + 1e-16), 1e-9)
            p = ad * asv
            iv = lax.bitcast_convert_type(p, jnp.int32)
            y = lax.bitcast_convert_type(
                jnp.full((16,), 0x5F3759DF, jnp.int32) - (iv >> 1), jnp.float32)
            for _i in range(3):
                y = y * (1.5 - 0.5 * p * y * y)
            a_v[r, pl.ds(0, 16)] = p * y
            return _
        lax.fori_loop(0, CHUNK, _row, None)
        pltpu.sync_copy(a_v, a_o.at[pl.ds(base, CHUNK)])
        return _

    lax.fori_loop(0, nchunks, _chunk, None)


@functools.cache
def _acoef_sc_kernel():
    return functools.partial(
        pl.kernel,
        mesh=plsc.VectorSubcoreMesh(core_axis_name="c", subcore_axis_name="s"),
        out_type=jax.ShapeDtypeStruct((E, 16), jnp.float32),
        scratch_types=[
            pltpu.VMEM((CHUNK,), jnp.int32),
            pltpu.VMEM((CHUNK,), jnp.int32),
            pltpu.VMEM((CHUNK, 16), jnp.float32),
            pltpu.VMEM((CHUNK, 128), jnp.float32),
            pltpu.VMEM((CHUNK, 128), jnp.float32),
            pltpu.VMEM((CHUNK, 16), jnp.float32),
            pltpu.SemaphoreType.DMA,
        ],
    )(_acoef_body)


def _hop_body(h2, a_t, src_g, dst_g, out2, sidx_v, didx_v, a_v, rows_v, zbuf,
              acc, sem):
    c = lax.axis_index("c")
    s = lax.axis_index("s")

    def _zrow(i, _):
        for j in range(8):
            zbuf[i, pl.ds(j * 16, 16)] = jnp.zeros((16,), jnp.float32)
        return _
    lax.fori_loop(0, 80, _zrow, None)
    for r in range(ROWS_PER_SUB // 80):
        pltpu.sync_copy(zbuf, acc.at[pl.ds(s * ROWS_PER_SUB + r * 80, 80)])
    plsc.subcore_barrier()

    nchunks = (NCHUNKS - s + NSUB - 1) // NSUB

    def _chunk(i, _):
        g = s + i * NSUB
        base = g * CHUNK
        pltpu.sync_copy(src_g.at[pl.ds(base, CHUNK)], sidx_v)
        pltpu.sync_copy(dst_g.at[pl.ds(base, CHUNK)], didx_v)
        pltpu.sync_copy(a_t.at[pl.ds(base, CHUNK)], a_v)
        for j in range(CHUNK // 16):
            sidx_v[pl.ds(j * 16, 16)] = sidx_v[pl.ds(j * 16, 16)] + c * NPAD
        pltpu.async_copy(h2.at[sidx_v], rows_v, sem).wait()

        def _scale(e, _):
            blk = a_v[e, pl.ds(0, 16)]
            s0 = blk.at[jnp.full((16,), 2 * c, jnp.int32)].get(
                mode="promise_in_bounds")
            s1 = blk.at[jnp.full((16,), 2 * c + 1, jnp.int32)].get(
                mode="promise_in_bounds")
            for j in range(4):
                rows_v[e, pl.ds(j * 16, 16)] = rows_v[e, pl.ds(j * 16, 16)] * s0
            for j in range(4, 8):
                rows_v[e, pl.ds(j * 16, 16)] = rows_v[e, pl.ds(j * 16, 16)] * s1
            return _
        lax.fori_loop(0, CHUNK, _scale, None)
        pltpu.sync_copy(rows_v, acc.at[didx_v], add=True)
        return _

    lax.fori_loop(0, nchunks, _chunk, None)
    plsc.subcore_barrier()
    pltpu.sync_copy(acc.at[pl.ds(s * ROWS_PER_SUB, ROWS_PER_SUB)],
                    out2.at[pl.ds(c * NPAD + s * ROWS_PER_SUB, ROWS_PER_SUB)])


@functools.cache
def _hop_sc_kernel():
    return functools.partial(
        pl.kernel,
        mesh=plsc.VectorSubcoreMesh(core_axis_name="c", subcore_axis_name="s"),
        out_type=jax.ShapeDtypeStruct((2 * NPAD, 128), jnp.float32),
        scratch_types=[
            pltpu.VMEM((CHUNK,), jnp.int32),
            pltpu.VMEM((CHUNK,), jnp.int32),
            pltpu.VMEM((CHUNK, 16), jnp.float32),
            pltpu.VMEM((CHUNK, 128), jnp.float32),
            pltpu.VMEM((80, 128), jnp.float32),
            pltpu.VMEM_SHARED((NPAD, 128), jnp.float32),
            pltpu.SemaphoreType.DMA,
        ],
    )(_hop_body)


def leaky_relu(x):
    return jnp.where(x >= 0, x, NEG * x)


def kernel(feat_src, edge_index, feat_edge, W_src, W_dst, b_dst, W_attn_src,
           W_attn_dst, W_attn_edge, scale, offset, position_emb, hop_attn_l,
           hop_attn_r):
    src = edge_index[0]
    dst = edge_index[1]

    w_cat = jnp.concatenate([W_src, W_dst, W_attn_src, W_attn_dst], axis=1)
    w_cat = jnp.pad(w_cat, ((0, 0), (0, 640 - w_cat.shape[1])))
    feat_pad = jnp.pad(feat_src, ((0, NPAD - N), (0, 0)))
    proj = _dense_proj(feat_pad, w_cat, block_rows=1024)

    w_e = jnp.pad(W_attn_edge, ((0, 0), (0, 16 - H)))
    ae16 = _dense_proj(feat_edge, w_e, block_rows=8000)

    asrc128 = jnp.pad(proj[:, 512:512 + H], ((0, NPAD - N), (0, 124)))
    adst128 = jnp.pad(proj[:, 512 + H:512 + 2 * H], ((0, NPAD - N), (0, 124)))
    ex16 = _edge_sc_kernel()(asrc128, adst128, ae16, src, dst)
    ones2 = jnp.ones((2 * NPAD, 128), jnp.float32)
    s2d = _hop_sc_kernel()(ones2, ex16, src, dst)
    s2s = _hop_sc_kernel()(ones2, ex16, dst, src)
    sd128 = jnp.pad(jnp.stack([s2d[:NPAD, 0], s2d[:NPAD, 64],
                               s2d[NPAD:, 0], s2d[NPAD:, 64]], axis=1),
                    ((0, 0), (0, 124)))
    ss128 = jnp.pad(jnp.stack([s2s[:NPAD, 0], s2s[:NPAD, 64],
                               s2s[NPAD:, 0], s2s[NPAD:, 64]], axis=1),
                    ((0, 0), (0, 124)))
    a16 = _acoef_sc_kernel()(ex16, sd128, ss128, src, dst)

    fc_pad = proj[:, :256]
    h2 = jnp.concatenate([fc_pad[:, :128], fc_pad[:, 128:]], axis=0)
    h2s = []
    for k in range(K):
        h2 = _hop_sc_kernel()(h2, a16, src, dst)
        h2s.append(h2)

    rst_pad = _combine_tc(
        proj, b_dst.reshape(1, 256), h2s,
        scale.reshape(K + 1, H * F), offset.reshape(K + 1, H * F),
        position_emb.reshape(K + 1, H * F),
        hop_attn_l.reshape(1, H * F), hop_attn_r.reshape(1, H * F))
    return rst_pad[:N].reshape(N, H, F)


# speedup vs baseline: 6.4560x; 1.2942x over previous
import functools

import jax
import jax.numpy as jnp
from jax import lax
from jax.experimental import pallas as pl
from jax.experimental.pallas import tpu as pltpu
from jax.experimental.pallas import tpu_sc as plsc

N = 10000
E = 160000
D = 256
DE = 16
H = 4
F = 64
K = 3
NEG = 0.2

NPAD = 10240
CHUNK = 128
NCHUNKS = E // CHUNK
NSUB = 16
ROWS_PER_SUB = NPAD // NSUB


def _proj_body(x_ref, w_ref, o_ref):
    o_ref[...] = jnp.dot(x_ref[...], w_ref[...],
                         preferred_element_type=jnp.float32)


def _dense_proj(x, w_cat, block_rows):
    R, Dk = x.shape
    C = w_cat.shape[1]
    grid = (R // block_rows,)
    return pl.pallas_call(
        _proj_body,
        grid=grid,
        in_specs=[
            pl.BlockSpec((block_rows, Dk), lambda i: (i, 0)),
            pl.BlockSpec((Dk, C), lambda i: (0, 0)),
        ],
        out_specs=pl.BlockSpec((block_rows, C), lambda i: (i, 0)),
        out_shape=jax.ShapeDtypeStruct((R, C), jnp.float32),
    )(x, w_cat)


def _combine_body(proj_ref, b_ref, h1l, h1h, h2l, h2h, h3l, h3h,
                  scl_ref, off_ref, pos_ref, al_ref, ar_ref, out_ref):
    f32 = jnp.float32
    hp = jax.lax.Precision.HIGHEST
    rows = lax.broadcasted_iota(jnp.int32, (256, 4), 0) // 64
    cols = lax.broadcasted_iota(jnp.int32, (256, 4), 1)
    m4 = jnp.where(rows == cols, 1.0, 0.0).astype(f32)
    m4avg = m4 * (1.0 / 64.0)
    e4 = m4.T

    def ft(h, k):
        mean4 = jnp.dot(h, m4avg, precision=hp, preferred_element_type=f32)
        ctr = h - jnp.dot(mean4, e4, precision=hp, preferred_element_type=f32)
        var4 = jnp.dot(ctr * ctr, m4avg, precision=hp, preferred_element_type=f32) + 1e-9
        rsE = jnp.dot(lax.rsqrt(var4), e4, precision=hp, preferred_element_type=f32)
        return ctr * rsE * scl_ref[k][None, :] + off_ref[k][None, :] + pos_ref[k][None, :]

    h0 = ft(proj_ref[:, 0:256], 0)
    lrow = al_ref[0][None, :]
    rrow = ar_ref[0][None, :]
    al4 = jnp.dot(h0 * lrow, m4, precision=hp, preferred_element_type=f32)
    fts = []
    w4s = []
    for k, (lo, hi) in enumerate(((h1l, h1h), (h2l, h2h), (h3l, h3h))):
        h = jnp.concatenate([lo[...], hi[...]], axis=1)
        f = ft(h, k + 1)
        fts.append(f)
        s = jnp.dot(f * rrow, m4, precision=hp, preferred_element_type=f32) + al4
        s = jnp.where(s >= 0, s, NEG * s)
        w4s.append(jnp.exp(s))
    den = w4s[0] + w4s[1] + w4s[2]
    acc = proj_ref[:, 256:512] + b_ref[...]
    for k in range(3):
        wE = jnp.dot(w4s[k] / den, e4, precision=hp, preferred_element_type=f32)
        acc = acc + fts[k] * wE
    out_ref[...] = acc


def _combine_tc(proj, b_dst, h2s, scl, off, pos, alf, arf):
    B = 1024
    grid = (NPAD // B,)
    hspec_lo = pl.BlockSpec((B, 128), lambda i: (i, 0))
    hspec_hi = pl.BlockSpec((B, 128), lambda i: (i + NPAD // B, 0))
    full = lambda shape: pl.BlockSpec(shape, lambda i: tuple(0 for _ in shape))
    return pl.pallas_call(
        _combine_body,
        grid=grid,
        in_specs=[
            pl.BlockSpec((B, 640), lambda i: (i, 0)),
            full((1, 256)),
            hspec_lo, hspec_hi, hspec_lo, hspec_hi, hspec_lo, hspec_hi,
            full((4, 256)), full((4, 256)), full((4, 256)),
            full((1, 256)), full((1, 256)),
        ],
        out_specs=pl.BlockSpec((B, 256), lambda i: (i, 0)),
        out_shape=jax.ShapeDtypeStruct((NPAD, 256), jnp.float32),
    )(proj, b_dst, h2s[0], h2s[0], h2s[1], h2s[1], h2s[2], h2s[2],
      scl, off, pos, alf, arf)


def _edge_body(asrc_t, adst_t, ae_t, src_g, dst_g, ex_o,
               sidx_v, didx_v, as_v, ad_v, ae_v, ex16_v, sem):
    c = lax.axis_index("c")
    s = lax.axis_index("s")
    w = c * NSUB + s
    nchunks = (NCHUNKS - w + 31) // 32

    def _chunk(i, _):
        g = w + i * 32
        base = g * CHUNK
        pltpu.sync_copy(src_g.at[pl.ds(base, CHUNK)], sidx_v)
        pltpu.sync_copy(dst_g.at[pl.ds(base, CHUNK)], didx_v)
        pltpu.sync_copy(ae_t.at[pl.ds(base, CHUNK)], ae_v)
        cp1 = pltpu.async_copy(asrc_t.at[sidx_v], as_v, sem)
        cp2 = pltpu.async_copy(adst_t.at[didx_v], ad_v, sem)
        cp1.wait()
        cp2.wait()

        def _row(r, _):
            x = as_v[r, pl.ds(0, 16)] + ad_v[r, pl.ds(0, 16)] + ae_v[r, pl.ds(0, 16)]
            x = jnp.where(x >= 0, x, NEG * x)
            ex16_v[r, pl.ds(0, 16)] = jnp.exp(x)
            return _
        lax.fori_loop(0, CHUNK, _row, None)
        pltpu.sync_copy(ex16_v, ex_o.at[pl.ds(base, CHUNK)])
        return _

    lax.fori_loop(0, nchunks, _chunk, None)


@functools.cache
def _edge_sc_kernel():
    return functools.partial(
        pl.kernel,
        mesh=plsc.VectorSubcoreMesh(core_axis_name="c", subcore_axis_name="s"),
        out_type=jax.ShapeDtypeStruct((E, 16), jnp.float32),
        scratch_types=[
            pltpu.VMEM((CHUNK,), jnp.int32),
            pltpu.VMEM((CHUNK,), jnp.int32),
            pltpu.VMEM((CHUNK, 128), jnp.float32),
            pltpu.VMEM((CHUNK, 128), jnp.float32),
            pltpu.VMEM((CHUNK, 16), jnp.float32),
            pltpu.VMEM((CHUNK, 16), jnp.float32),
            pltpu.SemaphoreType.DMA,
        ],
    )(_edge_body)


def _acoef_body(ex_t, sd_t, ss_t, src_g, dst_g, a_o,
                sidx_v, didx_v, ex_v, sd_v, ss_v, a_v, sem):
    c = lax.axis_index("c")
    s = lax.axis_index("s")
    w = c * NSUB + s
    nchunks = (NCHUNKS - w + 31) // 32

    def _chunk(i, _):
        g = w + i * 32
        base = g * CHUNK
        pltpu.sync_copy(src_g.at[pl.ds(base, CHUNK)], sidx_v)
        pltpu.sync_copy(dst_g.at[pl.ds(base, CHUNK)], didx_v)
        pltpu.sync_copy(ex_t.at[pl.ds(base, CHUNK)], ex_v)
        cp1 = pltpu.async_copy(sd_t.at[didx_v], sd_v, sem)
        cp2 = pltpu.async_copy(ss_t.at[sidx_v], ss_v, sem)
        cp1.wait()
        cp2.wait()

        def _row(r, _):
            exv = ex_v[r, pl.ds(0, 16)]
            ad = jnp.maximum(exv / (sd_v[r, pl.ds(0, 16)] + 1e-16), 1e-9)
            asv = jnp.maximum(exv / (ss_v[r, pl.ds(0, 16)] + 1e-16), 1e-9)
            p = ad * asv
            iv = lax.bitcast_convert_type(p, jnp.int32)
            y = lax.bitcast_convert_type(
                jnp.full((16,), 0x5F3759DF, jnp.int32) - (iv >> 1), jnp.float32)
            for _i in range(3):
                y = y * (1.5 - 0.5 * p * y * y)
            a_v[r, pl.ds(0, 16)] = p * y
            return _
        lax.fori_loop(0, CHUNK, _row, None)
        pltpu.sync_copy(a_v, a_o.at[pl.ds(base, CHUNK)])
        return _

    lax.fori_loop(0, nchunks, _chunk, None)


@functools.cache
def _acoef_sc_kernel():
    return functools.partial(
        pl.kernel,
        mesh=plsc.VectorSubcoreMesh(core_axis_name="c", subcore_axis_name="s"),
        out_type=jax.ShapeDtypeStruct((E, 16), jnp.float32),
        scratch_types=[
            pltpu.VMEM((CHUNK,), jnp.int32),
            pltpu.VMEM((CHUNK,), jnp.int32),
            pltpu.VMEM((CHUNK, 16), jnp.float32),
            pltpu.VMEM((CHUNK, 128), jnp.float32),
            pltpu.VMEM((CHUNK, 128), jnp.float32),
            pltpu.VMEM((CHUNK, 16), jnp.float32),
            pltpu.SemaphoreType.DMA,
        ],
    )(_acoef_body)


def _segsum_body(ex_t, idx2_g, out2, idx_v, ex_v, zbuf, acc, sem):
    c = lax.axis_index("c")
    s = lax.axis_index("s")

    def _zrow(i, _):
        zbuf[i, pl.ds(0, 16)] = jnp.zeros((16,), jnp.float32)
        return _
    lax.fori_loop(0, 80, _zrow, None)
    for r in range(ROWS_PER_SUB // 80):
        pltpu.sync_copy(zbuf, acc.at[pl.ds(s * ROWS_PER_SUB + r * 80, 80)])
    plsc.subcore_barrier()

    nchunks = (NCHUNKS - s + NSUB - 1) // NSUB

    def _chunk(i, _):
        g = s + i * NSUB
        base = g * CHUNK
        pltpu.sync_copy(idx2_g.at[pl.ds(c * E + base, CHUNK)], idx_v)
        pltpu.sync_copy(ex_t.at[pl.ds(base, CHUNK)], ex_v)
        pltpu.sync_copy(ex_v, acc.at[idx_v], add=True)
        return _

    lax.fori_loop(0, nchunks, _chunk, None)
    plsc.subcore_barrier()
    pltpu.sync_copy(acc.at[pl.ds(s * ROWS_PER_SUB, ROWS_PER_SUB)],
                    out2.at[pl.ds(c * NPAD + s * ROWS_PER_SUB, ROWS_PER_SUB)])


@functools.cache
def _segsum_sc_kernel():
    return functools.partial(
        pl.kernel,
        mesh=plsc.VectorSubcoreMesh(core_axis_name="c", subcore_axis_name="s"),
        out_type=jax.ShapeDtypeStruct((2 * NPAD, 16), jnp.float32),
        scratch_types=[
            pltpu.VMEM((CHUNK,), jnp.int32),
            pltpu.VMEM((CHUNK, 16), jnp.float32),
            pltpu.VMEM((80, 16), jnp.float32),
            pltpu.VMEM_SHARED((NPAD, 16), jnp.float32),
            pltpu.SemaphoreType.DMA,
        ],
    )(_segsum_body)


def _hop_body(h2, a_t, src_g, dst_g, out2, sidx_v, didx_v, a_v, rows_v, zbuf,
              acc, sem):
    c = lax.axis_index("c")
    s = lax.axis_index("s")

    def _zrow(i, _):
        for j in range(8):
            zbuf[i, pl.ds(j * 16, 16)] = jnp.zeros((16,), jnp.float32)
        return _
    lax.fori_loop(0, 80, _zrow, None)
    for r in range(ROWS_PER_SUB // 80):
        pltpu.sync_copy(zbuf, acc.at[pl.ds(s * ROWS_PER_SUB + r * 80, 80)])
    plsc.subcore_barrier()

    nchunks = (NCHUNKS - s + NSUB - 1) // NSUB

    def _chunk(i, _):
        g = s + i * NSUB
        base = g * CHUNK
        pltpu.sync_copy(src_g.at[pl.ds(base, CHUNK)], sidx_v)
        pltpu.sync_copy(dst_g.at[pl.ds(base, CHUNK)], didx_v)
        pltpu.sync_copy(a_t.at[pl.ds(base, CHUNK)], a_v)
        for j in range(CHUNK // 16):
            sidx_v[pl.ds(j * 16, 16)] = sidx_v[pl.ds(j * 16, 16)] + c * NPAD
        pltpu.async_copy(h2.at[sidx_v], rows_v, sem).wait()

        def _scale(e, _):
            blk = a_v[e, pl.ds(0, 16)]
            s0 = blk.at[jnp.full((16,), 2 * c, jnp.int32)].get(
                mode="promise_in_bounds")
            s1 = blk.at[jnp.full((16,), 2 * c + 1, jnp.int32)].get(
                mode="promise_in_bounds")
            for j in range(4):
                rows_v[e, pl.ds(j * 16, 16)] = rows_v[e, pl.ds(j * 16, 16)] * s0
            for j in range(4, 8):
                rows_v[e, pl.ds(j * 16, 16)] = rows_v[e, pl.ds(j * 16, 16)] * s1
            return _
        lax.fori_loop(0, CHUNK, _scale, None)
        pltpu.sync_copy(rows_v, acc.at[didx_v], add=True)
        return _

    lax.fori_loop(0, nchunks, _chunk, None)
    plsc.subcore_barrier()
    pltpu.sync_copy(acc.at[pl.ds(s * ROWS_PER_SUB, ROWS_PER_SUB)],
                    out2.at[pl.ds(c * NPAD + s * ROWS_PER_SUB, ROWS_PER_SUB)])


@functools.cache
def _hop_sc_kernel():
    return functools.partial(
        pl.kernel,
        mesh=plsc.VectorSubcoreMesh(core_axis_name="c", subcore_axis_name="s"),
        out_type=jax.ShapeDtypeStruct((2 * NPAD, 128), jnp.float32),
        scratch_types=[
            pltpu.VMEM((CHUNK,), jnp.int32),
            pltpu.VMEM((CHUNK,), jnp.int32),
            pltpu.VMEM((CHUNK, 16), jnp.float32),
            pltpu.VMEM((CHUNK, 128), jnp.float32),
            pltpu.VMEM((80, 128), jnp.float32),
            pltpu.VMEM_SHARED((NPAD, 128), jnp.float32),
            pltpu.SemaphoreType.DMA,
        ],
    )(_hop_body)


def leaky_relu(x):
    return jnp.where(x >= 0, x, NEG * x)


def kernel(feat_src, edge_index, feat_edge, W_src, W_dst, b_dst, W_attn_src,
           W_attn_dst, W_attn_edge, scale, offset, position_emb, hop_attn_l,
           hop_attn_r):
    src = edge_index[0]
    dst = edge_index[1]

    w_cat = jnp.concatenate([W_src, W_dst, W_attn_src, W_attn_dst], axis=1)
    w_cat = jnp.pad(w_cat, ((0, 0), (0, 640 - w_cat.shape[1])))
    feat_pad = jnp.pad(feat_src, ((0, NPAD - N), (0, 0)))
    proj = _dense_proj(feat_pad, w_cat, block_rows=1024)

    w_e = jnp.pad(W_attn_edge, ((0, 0), (0, 16 - H)))
    ae16 = _dense_proj(feat_edge, w_e, block_rows=8000)

    asrc128 = jnp.pad(proj[:, 512:512 + H], ((0, NPAD - N), (0, 124)))
    adst128 = jnp.pad(proj[:, 512 + H:512 + 2 * H], ((0, NPAD - N), (0, 124)))
    ex16 = _edge_sc_kernel()(asrc128, adst128, ae16, src, dst)
    idx2 = jnp.concatenate([dst, src])
    s216 = _segsum_sc_kernel()(ex16, idx2)
    sd128 = jnp.pad(s216[:NPAD], ((0, 0), (0, 112)))
    ss128 = jnp.pad(s216[NPAD:], ((0, 0), (0, 112)))
    a16 = _acoef_sc_kernel()(ex16, sd128, ss128, src, dst)

    fc_pad = proj[:, :256]
    h2 = jnp.concatenate([fc_pad[:, :128], fc_pad[:, 128:]], axis=0)
    h2s = []
    for k in range(K):
        h2 = _hop_sc_kernel()(h2, a16, src, dst)
        h2s.append(h2)

    rst_pad = _combine_tc(
        proj, b_dst.reshape(1, 256), h2s,
        scale.reshape(K + 1, H * F), offset.reshape(K + 1, H * F),
        position_emb.reshape(K + 1, H * F),
        hop_attn_l.reshape(1, H * F), hop_attn_r.reshape(1, H * F))
    return rst_pad[:N].reshape(N, H, F)
